# baseline (speedup 1.0000x reference)
import functools

import jax
import jax.numpy as jnp
from jax import lax
from jax.experimental import pallas as pl
from jax.experimental.pallas import tpu as pltpu
from jax.experimental.pallas import tpu_sc as plsc

_NC = 2
_NS = 16
_L = 16
_NW = _NC * _NS

_D = 128
_DC = _D // _L
_G = 64
_MARGIN = 1.0


def _sc_kernel_fn(B):
    BPW = B // _NW
    NCH = BPW // _G
    NGRP = _G // _L

    mesh = plsc.VectorSubcoreMesh(
        core_axis_name="c", subcore_axis_name="s",
        num_cores=_NC, num_subcores=_NS)

    def body(ph, pr, pt, nh, nr, nt, etab, rtab, wd, wg,
             o_dpos, o_pwdot, o_dneg, o_gdot,
             idx_h, idx_r, idx_t,
             hrows0, rrows0, trows0, hrows1, rrows1, trows1,
             wd_v, wg_v, od_v, ow_v, sem0, sem1):
        wid = lax.axis_index("s") * _NC + lax.axis_index("c")
        base = wid * BPW

        pltpu.sync_copy(wd, wd_v)
        pltpu.sync_copy(wg, wg_v)

        zero16 = jnp.zeros((_L,), jnp.float32)
        lanes = lax.iota(jnp.int32, _L)
        bufs = ((hrows0, rrows0, trows0), (hrows1, rrows1, trows1))
        sems = (sem0, sem1)

        def side(hi, ri, ti, w_v, o1, o2):
            pltpu.sync_copy(hi.at[pl.ds(base, BPW)], idx_h)
            pltpu.sync_copy(ri.at[pl.ds(base, BPW)], idx_r)
            pltpu.sync_copy(ti.at[pl.ds(base, BPW)], idx_t)

            def fire(c, b):
                off = c * _G
                hb, rb, tb = bufs[b]
                sem = sems[b]
                return (
                    pltpu.async_copy(etab.at[idx_h.at[pl.ds(off, _G)]],
                                     hb, sem),
                    pltpu.async_copy(rtab.at[idx_r.at[pl.ds(off, _G)]],
                                     rb, sem),
                    pltpu.async_copy(etab.at[idx_t.at[pl.ds(off, _G)]],
                                     tb, sem),
                )

            def compute(c, b):
                hb, rb, tb = bufs[b]

                def grp_body(g, _):
                    tri = g * _L + lanes

                    def dchunk(dc, carry):
                        accd, accw = carry
                        dbase = dc * _L
                        wh = w_v[pl.ds(dbase, _L)]
                        wr = w_v[pl.ds(_D + dbase, _L)]
                        wt = w_v[pl.ds(2 * _D + dbase, _L)]
                        for k in range(_L):
                            dv = jnp.full((_L,), dbase + k, jnp.int32)
                            hv = plsc.load_gather(hb, [tri, dv])
                            rv = plsc.load_gather(rb, [tri, dv])
                            tv = plsc.load_gather(tb, [tri, dv])
                            accd = accd + jnp.abs(hv + rv - tv)
                            accw = (accw + hv * wh[k] + rv * wr[k]
                                    + tv * wt[k])
                        return accd, accw

                    accd, accw = lax.fori_loop(
                        0, _DC, dchunk, (zero16, zero16))
                    s = c * _G + g * _L
                    od_v[pl.ds(s, _L)] = accd
                    ow_v[pl.ds(s, _L)] = accw
                    return 0

                lax.fori_loop(0, NGRP, grp_body, 0)

            pending = fire(0, 0)
            for c in range(NCH):
                nxt = None
                if c + 1 < NCH:
                    nxt = fire(c + 1, (c + 1) % 2)
                for cp in pending:
                    cp.wait()
                compute(c, c % 2)
                pending = nxt
            pltpu.sync_copy(od_v, o1.at[pl.ds(base, BPW)])
            pltpu.sync_copy(ow_v, o2.at[pl.ds(base, BPW)])

        side(ph, pr, pt, wd_v, o_dpos, o_pwdot)
        side(nh, nr, nt, wg_v, o_dneg, o_gdot)

    f32 = jnp.float32
    return pl.kernel(
        body,
        out_type=[jax.ShapeDtypeStruct((B,), f32) for _ in range(4)],
        mesh=mesh,
        compiler_params=pltpu.CompilerParams(needs_layout_passes=False),
        scratch_types=[
            pltpu.VMEM((BPW,), jnp.int32),
            pltpu.VMEM((BPW,), jnp.int32),
            pltpu.VMEM((BPW,), jnp.int32),
            pltpu.VMEM((_G, _D), f32),
            pltpu.VMEM((_G, _D), f32),
            pltpu.VMEM((_G, _D), f32),
            pltpu.VMEM((_G, _D), f32),
            pltpu.VMEM((_G, _D), f32),
            pltpu.VMEM((_G, _D), f32),
            pltpu.VMEM((3 * _D,), f32),
            pltpu.VMEM((3 * _D,), f32),
            pltpu.VMEM((BPW,), f32),
            pltpu.VMEM((BPW,), f32),
            pltpu.SemaphoreType.DMA,
            pltpu.SemaphoreType.DMA,
        ],
    )


def _tc_body(bg_ref, bd_ref, dpos_ref, pw_ref, dn_ref, gd_ref, out_ref):
    bg0 = bg_ref[0]
    bd0 = bd_ref[0]
    gd = gd_ref[...]
    g = 1.0 / (1.0 + jnp.exp(-(gd + bg0)))
    mask = (g > 0.5).astype(jnp.float32)
    m_sum = jnp.sum(dn_ref[...] * mask)
    c_sum = jnp.sum(mask)
    neg_scores = m_sum / c_sum
    x = dpos_ref[...] - _MARGIN
    ps = jnp.maximum(x, 0.0) + jnp.log(1.0 + jnp.exp(-jnp.abs(x)))
    dout = 1.0 / (1.0 + jnp.exp(-(pw_ref[...] + bd0)))
    out_ref[...] = jnp.sum(dout * (ps + neg_scores)).reshape(1, 1)


def _tc_call(bg0, bd0, dpos, pwdot, dneg, gdot):
    n = dpos.shape[0]
    shape2 = (n // 128, 128)
    return pl.pallas_call(
        _tc_body,
        out_shape=jax.ShapeDtypeStruct((1, 1), jnp.float32),
        in_specs=[
            pl.BlockSpec(memory_space=pltpu.SMEM),
            pl.BlockSpec(memory_space=pltpu.SMEM),
            pl.BlockSpec(memory_space=pltpu.VMEM),
            pl.BlockSpec(memory_space=pltpu.VMEM),
            pl.BlockSpec(memory_space=pltpu.VMEM),
            pl.BlockSpec(memory_space=pltpu.VMEM),
        ],
        out_specs=pl.BlockSpec(memory_space=pltpu.VMEM),
    )(bg0, bd0, dpos.reshape(shape2), pwdot.reshape(shape2),
      dneg.reshape(shape2), gdot.reshape(shape2))


def kernel(positive_triples, negative_triples, entity_table, relation_table,
           Wg, bg, Wd, bd):
    B = positive_triples.shape[0]
    ph = positive_triples[:, 0]
    pr = positive_triples[:, 1]
    pt = positive_triples[:, 2]
    nh = negative_triples[:, 0]
    nr = negative_triples[:, 1]
    nt = negative_triples[:, 2]
    wd = Wd[:, 0]
    wg = Wg[:, 0]

    dpos, pwdot, dneg, gdot = _sc_kernel_fn(B)(
        ph, pr, pt, nh, nr, nt, entity_table, relation_table, wd, wg)

    out = _tc_call(bg[:1], bd[:1], dpos, pwdot, dneg, gdot)
    return out[0, 0]


# speedup vs baseline: 3.5338x; 3.5338x over previous
import functools

import jax
import jax.numpy as jnp
from jax import lax
from jax.experimental import pallas as pl
from jax.experimental.pallas import tpu as pltpu
from jax.experimental.pallas import tpu_sc as plsc

_NC = 2
_NS = 16
_L = 16
_NW = _NC * _NS

_D = 128
_DC = _D // _L
_G = 64
_MARGIN = 1.0


def _sc_kernel_fn(B):
    BPW = B // _NW
    NCH = BPW // _G
    NGRP = _G // _L

    mesh = plsc.VectorSubcoreMesh(
        core_axis_name="c", subcore_axis_name="s",
        num_cores=_NC, num_subcores=_NS)

    def body(ph, pr, pt, nh, nr, nt, etab, rtab, wd, wg,
             o_dpos, o_pwdot, o_dneg, o_gdot,
             idx_h, idx_r, idx_t,
             hrows0, rrows0, trows0, hrows1, rrows1, trows1,
             wd_v, wg_v, od_v, ow_v, sem0, sem1):
        wid = lax.axis_index("s") * _NC + lax.axis_index("c")
        base = wid * BPW

        pltpu.sync_copy(wd, wd_v)
        pltpu.sync_copy(wg, wg_v)

        zero16 = jnp.zeros((_L,), jnp.float32)
        lanes = lax.iota(jnp.int32, _L)
        bufs = ((hrows0, rrows0, trows0), (hrows1, rrows1, trows1))
        sems = (sem0, sem1)

        def side(hi, ri, ti, w_v, o1, o2):
            pltpu.sync_copy(hi.at[pl.ds(base, BPW)], idx_h)
            pltpu.sync_copy(ri.at[pl.ds(base, BPW)], idx_r)
            pltpu.sync_copy(ti.at[pl.ds(base, BPW)], idx_t)
            wch = [(w_v[pl.ds(cc * _L, _L)],
                    w_v[pl.ds(_D + cc * _L, _L)],
                    w_v[pl.ds(2 * _D + cc * _L, _L)]) for cc in range(_DC)]

            def fire(c, b):
                off = c * _G
                hb, rb, tb = bufs[b]
                sem = sems[b]
                return (
                    pltpu.async_copy(etab.at[idx_h.at[pl.ds(off, _G)]],
                                     hb, sem),
                    pltpu.async_copy(rtab.at[idx_r.at[pl.ds(off, _G)]],
                                     rb, sem),
                    pltpu.async_copy(etab.at[idx_t.at[pl.ds(off, _G)]],
                                     tb, sem),
                )

            def compute(c, b):
                hb, rb, tb = bufs[b]

                def grp_body(g, _):
                    def tri_body(j, carry):
                        od16, ow16 = carry
                        row = g * _L + j
                        accd = zero16
                        accw = zero16
                        for cc in range(_DC):
                            hv = hb[row, pl.ds(cc * _L, _L)]
                            rv = rb[row, pl.ds(cc * _L, _L)]
                            tv = tb[row, pl.ds(cc * _L, _L)]
                            accd = accd + jnp.abs(hv + rv - tv)
                            wh, wr, wt = wch[cc]
                            accw = accw + hv * wh + rv * wr + tv * wt
                        dsum = jnp.sum(accd)
                        wsum = jnp.sum(accw)
                        sel = lanes == j
                        od16 = jnp.where(sel, dsum, od16)
                        ow16 = jnp.where(sel, wsum, ow16)
                        return od16, ow16

                    od16, ow16 = lax.fori_loop(
                        0, _L, tri_body, (zero16, zero16), unroll=2)
                    s = c * _G + g * _L
                    od_v[pl.ds(s, _L)] = od16
                    ow_v[pl.ds(s, _L)] = ow16
                    return 0

                lax.fori_loop(0, NGRP, grp_body, 0)

            pending = fire(0, 0)
            for c in range(NCH):
                nxt = None
                if c + 1 < NCH:
                    nxt = fire(c + 1, (c + 1) % 2)
                for cp in pending:
                    cp.wait()
                compute(c, c % 2)
                pending = nxt
            pltpu.sync_copy(od_v, o1.at[pl.ds(base, BPW)])
            pltpu.sync_copy(ow_v, o2.at[pl.ds(base, BPW)])

        side(ph, pr, pt, wd_v, o_dpos, o_pwdot)
        side(nh, nr, nt, wg_v, o_dneg, o_gdot)

    f32 = jnp.float32
    return pl.kernel(
        body,
        out_type=[jax.ShapeDtypeStruct((B,), f32) for _ in range(4)],
        mesh=mesh,
        compiler_params=pltpu.CompilerParams(needs_layout_passes=False),
        scratch_types=[
            pltpu.VMEM((BPW,), jnp.int32),
            pltpu.VMEM((BPW,), jnp.int32),
            pltpu.VMEM((BPW,), jnp.int32),
            pltpu.VMEM((_G, _D), f32),
            pltpu.VMEM((_G, _D), f32),
            pltpu.VMEM((_G, _D), f32),
            pltpu.VMEM((_G, _D), f32),
            pltpu.VMEM((_G, _D), f32),
            pltpu.VMEM((_G, _D), f32),
            pltpu.VMEM((3 * _D,), f32),
            pltpu.VMEM((3 * _D,), f32),
            pltpu.VMEM((BPW,), f32),
            pltpu.VMEM((BPW,), f32),
            pltpu.SemaphoreType.DMA,
            pltpu.SemaphoreType.DMA,
        ],
    )


def _tc_body(bg_ref, bd_ref, dpos_ref, pw_ref, dn_ref, gd_ref, out_ref):
    bg0 = bg_ref[0]
    bd0 = bd_ref[0]
    gd = gd_ref[...]
    g = 1.0 / (1.0 + jnp.exp(-(gd + bg0)))
    mask = (g > 0.5).astype(jnp.float32)
    m_sum = jnp.sum(dn_ref[...] * mask)
    c_sum = jnp.sum(mask)
    neg_scores = m_sum / c_sum
    x = dpos_ref[...] - _MARGIN
    ps = jnp.maximum(x, 0.0) + jnp.log(1.0 + jnp.exp(-jnp.abs(x)))
    dout = 1.0 / (1.0 + jnp.exp(-(pw_ref[...] + bd0)))
    out_ref[...] = jnp.sum(dout * (ps + neg_scores)).reshape(1, 1)


def _tc_call(bg0, bd0, dpos, pwdot, dneg, gdot):
    n = dpos.shape[0]
    shape2 = (n // 128, 128)
    return pl.pallas_call(
        _tc_body,
        out_shape=jax.ShapeDtypeStruct((1, 1), jnp.float32),
        in_specs=[
            pl.BlockSpec(memory_space=pltpu.SMEM),
            pl.BlockSpec(memory_space=pltpu.SMEM),
            pl.BlockSpec(memory_space=pltpu.VMEM),
            pl.BlockSpec(memory_space=pltpu.VMEM),
            pl.BlockSpec(memory_space=pltpu.VMEM),
            pl.BlockSpec(memory_space=pltpu.VMEM),
        ],
        out_specs=pl.BlockSpec(memory_space=pltpu.VMEM),
    )(bg0, bd0, dpos.reshape(shape2), pwdot.reshape(shape2),
      dneg.reshape(shape2), gdot.reshape(shape2))


def kernel(positive_triples, negative_triples, entity_table, relation_table,
           Wg, bg, Wd, bd):
    B = positive_triples.shape[0]
    ph = positive_triples[:, 0]
    pr = positive_triples[:, 1]
    pt = positive_triples[:, 2]
    nh = negative_triples[:, 0]
    nr = negative_triples[:, 1]
    nt = negative_triples[:, 2]
    wd = Wd[:, 0]
    wg = Wg[:, 0]

    dpos, pwdot, dneg, gdot = _sc_kernel_fn(B)(
        ph, pr, pt, nh, nr, nt, entity_table, relation_table, wd, wg)

    out = _tc_call(bg[:1], bd[:1], dpos, pwdot, dneg, gdot)
    return out[0, 0]


# speedup vs baseline: 3.8630x; 1.0931x over previous
import functools

import jax
import jax.numpy as jnp
from jax import lax
from jax.experimental import pallas as pl
from jax.experimental.pallas import tpu as pltpu
from jax.experimental.pallas import tpu_sc as plsc

_NC = 2
_NS = 16
_L = 16
_NW = _NC * _NS

_D = 128
_DC = _D // _L
_G = 64
_MARGIN = 1.0


def _sc_kernel_fn(B):
    BPW = B // _NW
    NCH = BPW // _G
    NGRP = _G // _L

    mesh = plsc.VectorSubcoreMesh(
        core_axis_name="c", subcore_axis_name="s",
        num_cores=_NC, num_subcores=_NS)

    def body(ph, pr, pt, nh, nr, nt, etab, rtab, wd, wg,
             o_dpos, o_pwdot, o_dneg, o_gdot,
             idx_h, idx_r, idx_t,
             hrows0, rrows0, trows0, hrows1, rrows1, trows1,
             wd_v, wg_v, od_v, ow_v, sem0, sem1):
        wid = lax.axis_index("s") * _NC + lax.axis_index("c")
        base = wid * BPW

        pltpu.sync_copy(wd, wd_v)
        pltpu.sync_copy(wg, wg_v)

        zero16 = jnp.zeros((_L,), jnp.float32)
        lanes = lax.iota(jnp.int32, _L)
        bufs = ((hrows0, rrows0, trows0), (hrows1, rrows1, trows1))
        sems = (sem0, sem1)

        def side(hi, ri, ti, w_v, o1, o2):
            pltpu.sync_copy(hi.at[pl.ds(base, BPW)], idx_h)
            pltpu.sync_copy(ri.at[pl.ds(base, BPW)], idx_r)
            pltpu.sync_copy(ti.at[pl.ds(base, BPW)], idx_t)
            wch = [(w_v[pl.ds(cc * _L, _L)],
                    w_v[pl.ds(_D + cc * _L, _L)],
                    w_v[pl.ds(2 * _D + cc * _L, _L)]) for cc in range(_DC)]

            def fire(c, b):
                off = c * _G
                hb, rb, tb = bufs[b]
                sem = sems[b]
                return (
                    pltpu.async_copy(etab.at[idx_h.at[pl.ds(off, _G)]],
                                     hb, sem),
                    pltpu.async_copy(rtab.at[idx_r.at[pl.ds(off, _G)]],
                                     rb, sem),
                    pltpu.async_copy(etab.at[idx_t.at[pl.ds(off, _G)]],
                                     tb, sem),
                )

            def compute(c, b):
                hb, rb, tb = bufs[b]

                def grp_body(g, _):
                    def tri_body(j, carry):
                        od16, ow16 = carry
                        row = g * _L + j
                        accd = zero16
                        accw = zero16
                        for cc in range(_DC):
                            hv = hb[row, pl.ds(cc * _L, _L)]
                            rv = rb[row, pl.ds(cc * _L, _L)]
                            tv = tb[row, pl.ds(cc * _L, _L)]
                            accd = accd + jnp.abs(hv + rv - tv)
                            wh, wr, wt = wch[cc]
                            accw = accw + hv * wh + rv * wr + tv * wt
                        dsum = jnp.sum(accd)
                        wsum = jnp.sum(accw)
                        sel = lanes == j
                        od16 = jnp.where(sel, dsum, od16)
                        ow16 = jnp.where(sel, wsum, ow16)
                        return od16, ow16

                    od16, ow16 = lax.fori_loop(
                        0, _L, tri_body, (zero16, zero16))
                    s = c * _G + g * _L
                    od_v[pl.ds(s, _L)] = od16
                    ow_v[pl.ds(s, _L)] = ow16
                    return 0

                lax.fori_loop(0, NGRP, grp_body, 0)

            pending = fire(0, 0)
            for c in range(NCH):
                nxt = None
                if c + 1 < NCH:
                    nxt = fire(c + 1, (c + 1) % 2)
                for cp in pending:
                    cp.wait()
                compute(c, c % 2)
                pending = nxt
            pltpu.sync_copy(od_v, o1.at[pl.ds(base, BPW)])
            pltpu.sync_copy(ow_v, o2.at[pl.ds(base, BPW)])

        side(ph, pr, pt, wd_v, o_dpos, o_pwdot)
        side(nh, nr, nt, wg_v, o_dneg, o_gdot)

    f32 = jnp.float32
    return pl.kernel(
        body,
        out_type=[jax.ShapeDtypeStruct((B,), f32) for _ in range(4)],
        mesh=mesh,
        compiler_params=pltpu.CompilerParams(needs_layout_passes=False),
        scratch_types=[
            pltpu.VMEM((BPW,), jnp.int32),
            pltpu.VMEM((BPW,), jnp.int32),
            pltpu.VMEM((BPW,), jnp.int32),
            pltpu.VMEM((_G, _D), f32),
            pltpu.VMEM((_G, _D), f32),
            pltpu.VMEM((_G, _D), f32),
            pltpu.VMEM((_G, _D), f32),
            pltpu.VMEM((_G, _D), f32),
            pltpu.VMEM((_G, _D), f32),
            pltpu.VMEM((3 * _D,), f32),
            pltpu.VMEM((3 * _D,), f32),
            pltpu.VMEM((BPW,), f32),
            pltpu.VMEM((BPW,), f32),
            pltpu.SemaphoreType.DMA,
            pltpu.SemaphoreType.DMA,
        ],
    )


def _tc_body(bg_ref, bd_ref, dpos_ref, pw_ref, dn_ref, gd_ref, out_ref):
    bg0 = bg_ref[0]
    bd0 = bd_ref[0]
    gd = gd_ref[...]
    g = 1.0 / (1.0 + jnp.exp(-(gd + bg0)))
    mask = (g > 0.5).astype(jnp.float32)
    m_sum = jnp.sum(dn_ref[...] * mask)
    c_sum = jnp.sum(mask)
    neg_scores = m_sum / c_sum
    x = dpos_ref[...] - _MARGIN
    ps = jnp.maximum(x, 0.0) + jnp.log(1.0 + jnp.exp(-jnp.abs(x)))
    dout = 1.0 / (1.0 + jnp.exp(-(pw_ref[...] + bd0)))
    out_ref[...] = jnp.sum(dout * (ps + neg_scores)).reshape(1, 1)


def _tc_call(bg0, bd0, dpos, pwdot, dneg, gdot):
    n = dpos.shape[0]
    shape2 = (n // 128, 128)
    return pl.pallas_call(
        _tc_body,
        out_shape=jax.ShapeDtypeStruct((1, 1), jnp.float32),
        in_specs=[
            pl.BlockSpec(memory_space=pltpu.SMEM),
            pl.BlockSpec(memory_space=pltpu.SMEM),
            pl.BlockSpec(memory_space=pltpu.VMEM),
            pl.BlockSpec(memory_space=pltpu.VMEM),
            pl.BlockSpec(memory_space=pltpu.VMEM),
            pl.BlockSpec(memory_space=pltpu.VMEM),
        ],
        out_specs=pl.BlockSpec(memory_space=pltpu.VMEM),
    )(bg0, bd0, dpos.reshape(shape2), pwdot.reshape(shape2),
      dneg.reshape(shape2), gdot.reshape(shape2))


def kernel(positive_triples, negative_triples, entity_table, relation_table,
           Wg, bg, Wd, bd):
    B = positive_triples.shape[0]
    ph = positive_triples[:, 0]
    pr = positive_triples[:, 1]
    pt = positive_triples[:, 2]
    nh = negative_triples[:, 0]
    nr = negative_triples[:, 1]
    nt = negative_triples[:, 2]
    wd = Wd[:, 0]
    wg = Wg[:, 0]

    dpos, pwdot, dneg, gdot = _sc_kernel_fn(B)(
        ph, pr, pt, nh, nr, nt, entity_table, relation_table, wd, wg)

    out = _tc_call(bg[:1], bd[:1], dpos, pwdot, dneg, gdot)
    return out[0, 0]
